# baseline (speedup 1.0000x reference)
import functools

import jax
import jax.numpy as jnp
from jax import lax
from jax.experimental import pallas as pl
from jax.experimental.pallas import tpu as pltpu
from jax.experimental.pallas import tpu_sc as plsc

ROWS, COLS = 16384, 1024
NC, NS = 2, 16
NW = NC * NS
RPW = ROWS // NW
LANES = 16
CHUNK = 8
NCHUNK = RPW // CHUNK
NBUF = 4
NITER = NCHUNK // NBUF
VPR = COLS // LANES


def _splat_lane(fvec, lane):
    return lax.gather(
        fvec,
        jnp.full((LANES, 1), lane, jnp.int32),
        lax.GatherDimensionNumbers(
            offset_dims=(), collapsed_slice_dims=(0,), start_index_map=(0,)),
        (1,),
        mode=lax.GatherScatterMode.PROMISE_IN_BOUNDS)


def _sc_body(data_hbm, fac_hbm, out_hbm, in_bufs, out_bufs, fac_v,
             in_sem0, in_sem1, in_sem2, in_sem3,
             out_sem0, out_sem1, out_sem2, out_sem3):
    in_sems = (in_sem0, in_sem1, in_sem2, in_sem3)
    out_sems = (out_sem0, out_sem1, out_sem2, out_sem3)
    wid = lax.axis_index("s") * NC + lax.axis_index("c")
    pltpu.sync_copy(fac_hbm.at[pl.ds(0, ROWS)], fac_v.at[pl.ds(0, ROWS)])

    def in_copy(b, c):
        rbase = (c * NW + wid) * CHUNK
        return pltpu.make_async_copy(
            data_hbm.at[pl.ds(rbase, CHUNK), :], in_bufs.at[b], in_sems[b])

    def out_copy(b, c):
        rbase = (c * NW + wid) * CHUNK
        return pltpu.make_async_copy(
            out_bufs.at[b], out_hbm.at[pl.ds(rbase, CHUNK), :], out_sems[b])

    def compute(b, c):
        fvec = 1.0 + fac_v[pl.ds((c * NW + wid) * CHUNK, LANES)]

        @plsc.parallel_loop(0, CHUNK)
        def _rows(r):
            fsplat = _splat_lane(fvec, r)

            @plsc.parallel_loop(0, COLS, step=LANES, unroll=8)
            def _vecs(o):
                sl = pl.ds(o, LANES)
                out_bufs[b, r, sl] = in_bufs[b, r, sl] * fsplat

    for b in range(NBUF):
        in_copy(b, b).start()

    def iter_body(i, _):
        for b in range(NBUF):
            c = i * NBUF + b
            in_copy(b, c).wait()

            @pl.when(i > 0)
            def _():
                out_copy(b, c - NBUF).wait()

            compute(b, c)

            @pl.when(i < NITER - 1)
            def _():
                in_copy(b, c + NBUF).start()

            out_copy(b, c).start()
        return 0

    lax.fori_loop(0, NITER, iter_body, 0)

    for b in range(NBUF):
        out_copy(b, NCHUNK - NBUF + b).wait()


_sc_call = functools.partial(
    pl.kernel,
    out_type=jax.ShapeDtypeStruct((ROWS, COLS), jnp.float32),
    mesh=plsc.VectorSubcoreMesh(core_axis_name="c", subcore_axis_name="s"),
    scratch_types=[
        pltpu.VMEM((NBUF, CHUNK, COLS), jnp.float32),
        pltpu.VMEM((NBUF, CHUNK, COLS), jnp.float32),
        pltpu.VMEM((ROWS + LANES,), jnp.float32),
        pltpu.SemaphoreType.DMA,
        pltpu.SemaphoreType.DMA,
        pltpu.SemaphoreType.DMA,
        pltpu.SemaphoreType.DMA,
        pltpu.SemaphoreType.DMA,
        pltpu.SemaphoreType.DMA,
        pltpu.SemaphoreType.DMA,
        pltpu.SemaphoreType.DMA,
    ],
)(_sc_body)


def kernel(data, applied_mask):
    fac = applied_mask.astype(jnp.float32)
    return _sc_call(data, fac)


# speedup vs baseline: 1.0343x; 1.0343x over previous
import functools

import jax
import jax.numpy as jnp
from jax import lax
from jax.experimental import pallas as pl
from jax.experimental.pallas import tpu as pltpu
from jax.experimental.pallas import tpu_sc as plsc

ROWS, COLS = 16384, 1024
NC, NS = 2, 16
NW = NC * NS
RPW = ROWS // NW
LANES = 16
CHUNK = 4
NCHUNK = RPW // CHUNK
NBUF = 8
NITER = NCHUNK // NBUF
VPR = COLS // LANES


def _splat_lane(fvec, lane):
    return lax.gather(
        fvec,
        jnp.full((LANES, 1), lane, jnp.int32),
        lax.GatherDimensionNumbers(
            offset_dims=(), collapsed_slice_dims=(0,), start_index_map=(0,)),
        (1,),
        mode=lax.GatherScatterMode.PROMISE_IN_BOUNDS)


def _sc_body(data_hbm, fac_hbm, out_hbm, in_bufs, out_bufs, fac_v,
             in_sem0, in_sem1, in_sem2, in_sem3,
             in_sem4, in_sem5, in_sem6, in_sem7,
             out_sem0, out_sem1, out_sem2, out_sem3,
             out_sem4, out_sem5, out_sem6, out_sem7):
    in_sems = (in_sem0, in_sem1, in_sem2, in_sem3,
               in_sem4, in_sem5, in_sem6, in_sem7)
    out_sems = (out_sem0, out_sem1, out_sem2, out_sem3,
                out_sem4, out_sem5, out_sem6, out_sem7)
    wid = lax.axis_index("s") * NC + lax.axis_index("c")
    base = wid * RPW
    pltpu.sync_copy(fac_hbm.at[pl.ds(base, RPW)], fac_v.at[pl.ds(0, RPW)])

    def in_copy(b, c):
        rbase = base + c * CHUNK
        return pltpu.make_async_copy(
            data_hbm.at[pl.ds(rbase, CHUNK), :], in_bufs.at[b], in_sems[b])

    def out_copy(b, c):
        rbase = base + c * CHUNK
        return pltpu.make_async_copy(
            out_bufs.at[b], out_hbm.at[pl.ds(rbase, CHUNK), :], out_sems[b])

    def compute(b, c):
        fvec = 1.0 + fac_v[pl.ds(c * CHUNK, LANES)]

        @plsc.parallel_loop(0, CHUNK)
        def _rows(r):
            fsplat = _splat_lane(fvec, r)

            @plsc.parallel_loop(0, COLS, step=LANES, unroll=8)
            def _vecs(o):
                sl = pl.ds(o, LANES)
                out_bufs[b, r, sl] = in_bufs[b, r, sl] * fsplat

    for b in range(NBUF):
        in_copy(b, b).start()

    def iter_body(i, _):
        for b in range(NBUF):
            c = i * NBUF + b
            in_copy(b, c).wait()

            @pl.when(i > 0)
            def _():
                out_copy(b, c - NBUF).wait()

            compute(b, c)

            @pl.when(i < NITER - 1)
            def _():
                in_copy(b, c + NBUF).start()

            out_copy(b, c).start()
        return 0

    lax.fori_loop(0, NITER, iter_body, 0)

    for b in range(NBUF):
        out_copy(b, NCHUNK - NBUF + b).wait()


_sc_call = functools.partial(
    pl.kernel,
    out_type=jax.ShapeDtypeStruct((ROWS, COLS), jnp.float32),
    mesh=plsc.VectorSubcoreMesh(core_axis_name="c", subcore_axis_name="s"),
    scratch_types=[
        pltpu.VMEM((NBUF, CHUNK, COLS), jnp.float32),
        pltpu.VMEM((NBUF, CHUNK, COLS), jnp.float32),
        pltpu.VMEM((RPW + LANES,), jnp.float32),
        pltpu.SemaphoreType.DMA,
        pltpu.SemaphoreType.DMA,
        pltpu.SemaphoreType.DMA,
        pltpu.SemaphoreType.DMA,
        pltpu.SemaphoreType.DMA,
        pltpu.SemaphoreType.DMA,
        pltpu.SemaphoreType.DMA,
        pltpu.SemaphoreType.DMA,
        pltpu.SemaphoreType.DMA,
        pltpu.SemaphoreType.DMA,
        pltpu.SemaphoreType.DMA,
        pltpu.SemaphoreType.DMA,
        pltpu.SemaphoreType.DMA,
        pltpu.SemaphoreType.DMA,
        pltpu.SemaphoreType.DMA,
        pltpu.SemaphoreType.DMA,
    ],
)(_sc_body)


def kernel(data, applied_mask):
    fac = applied_mask.astype(jnp.float32)
    return _sc_call(data, fac)
